# Initial kernel scaffold; baseline (speedup 1.0000x reference)
import jax
import jax.numpy as jnp
from jax.experimental import pallas as pl


def kernel(hidden_states, gate_weight):
    raise NotImplementedError("write your pallas kernel here")


# speedup vs baseline: 1.2939x; 1.2939x over previous
import jax
import jax.numpy as jnp
from jax.experimental import pallas as pl
from jax.experimental.pallas import tpu as pltpu

TOPK = 8


def _router_body(x_ref, wt_ref, idx_ref, w_ref, logits_ref, loss_ref,
                 imp_acc, load_acc, *, n_tokens, grid):
    pid = pl.program_id(0)
    e_dim = wt_ref.shape[1]
    bt = x_ref.shape[0]

    logits = jnp.dot(x_ref[...], wt_ref[...], preferred_element_type=jnp.float32)
    logits_ref[...] = logits

    m = jnp.max(logits, axis=1, keepdims=True)
    ex = jnp.exp(logits - m)
    s = jnp.sum(ex, axis=1, keepdims=True)
    probs = ex / s

    lane = jax.lax.broadcasted_iota(jnp.int32, (bt, e_dim), 1)
    work = logits
    sel = jnp.zeros((bt, e_dim), jnp.float32)
    top_v = []
    top_i = []
    for _ in range(TOPK):
        cm = jnp.max(work, axis=1, keepdims=True)
        ci = jnp.min(jnp.where(work == cm, lane, e_dim), axis=1, keepdims=True)
        chosen = lane == ci
        sel = jnp.where(chosen, 1.0, sel)
        work = jnp.where(chosen, -jnp.inf, work)
        top_v.append(cm)
        top_i.append(ci)
    vals = jnp.concatenate(top_v, axis=1)
    idxs = jnp.concatenate(top_i, axis=1)

    ev = jnp.exp(vals - m)
    w_ref[...] = ev / jnp.sum(ev, axis=1, keepdims=True)
    idx_ref[...] = idxs

    @pl.when(pid == 0)
    def _init():
        imp_acc[...] = jnp.zeros_like(imp_acc)
        load_acc[...] = jnp.zeros_like(load_acc)

    imp_acc[...] += jnp.sum(probs, axis=0, keepdims=True)
    load_acc[...] += jnp.sum(sel, axis=0, keepdims=True)

    @pl.when(pid == grid - 1)
    def _finish():
        imp = imp_acc[...] / jnp.float32(n_tokens)
        load = load_acc[...] / jnp.float32(n_tokens * TOPK)
        loss_ref[...] = jnp.float32(e_dim) * jnp.sum(
            imp * load, axis=(0, 1), keepdims=True)


def kernel(hidden_states, gate_weight):
    n_tokens, d_model = hidden_states.shape
    e_dim = gate_weight.shape[0]
    bt = min(512, n_tokens)
    grid = n_tokens // bt

    body = lambda *refs: _router_body(*refs, n_tokens=n_tokens, grid=grid)
    out = pl.pallas_call(
        body,
        grid=(grid,),
        in_specs=[
            pl.BlockSpec((bt, d_model), lambda i: (i, 0)),
            pl.BlockSpec((d_model, e_dim), lambda i: (0, 0)),
        ],
        out_specs=[
            pl.BlockSpec((bt, TOPK), lambda i: (i, 0)),
            pl.BlockSpec((bt, TOPK), lambda i: (i, 0)),
            pl.BlockSpec((bt, e_dim), lambda i: (i, 0)),
            pl.BlockSpec((1, 1), lambda i: (0, 0)),
        ],
        out_shape=[
            jax.ShapeDtypeStruct((n_tokens, TOPK), jnp.int32),
            jax.ShapeDtypeStruct((n_tokens, TOPK), jnp.float32),
            jax.ShapeDtypeStruct((n_tokens, e_dim), jnp.float32),
            jax.ShapeDtypeStruct((1, 1), jnp.float32),
        ],
        scratch_shapes=[
            pltpu.VMEM((1, e_dim), jnp.float32),
            pltpu.VMEM((1, e_dim), jnp.float32),
        ],
        compiler_params=pltpu.CompilerParams(
            dimension_semantics=("arbitrary",),
        ),
    )(hidden_states, gate_weight.T)
    idxs, weights, logits, loss = out
    return idxs, weights, logits, loss[0, 0]


# speedup vs baseline: 1.4864x; 1.1488x over previous
import jax
import jax.numpy as jnp
from jax.experimental import pallas as pl
from jax.experimental.pallas import tpu as pltpu

TOPK = 8


def _router_body(x_ref, wt_ref, idx_ref, w_ref, logits_ref, loss_ref,
                 imp_acc, load_acc, *, n_tokens, grid):
    pid = pl.program_id(0)
    e_dim = wt_ref.shape[1]
    bt = x_ref.shape[0]

    logits = jnp.dot(x_ref[...], wt_ref[...], preferred_element_type=jnp.float32)
    logits_ref[...] = logits

    m = jnp.max(logits, axis=1, keepdims=True)
    ex = jnp.exp(logits - m)
    s = jnp.sum(ex, axis=1, keepdims=True)
    probs = ex / s

    lane = jax.lax.broadcasted_iota(jnp.int32, (bt, e_dim), 1)
    work = logits
    sel = jnp.zeros((bt, e_dim), jnp.float32)
    top_v = []
    top_i = []
    for _ in range(TOPK):
        cm = jnp.max(work, axis=1, keepdims=True)
        ci = jnp.min(jnp.where(work == cm, lane, e_dim), axis=1, keepdims=True)
        chosen = lane == ci
        sel = jnp.where(chosen, 1.0, sel)
        work = jnp.where(chosen, -jnp.inf, work)
        top_v.append(cm)
        top_i.append(ci)
    vals = jnp.concatenate(top_v, axis=1)
    idxs = jnp.concatenate(top_i, axis=1)

    ev = jnp.exp(vals - m)
    w_ref[...] = ev / jnp.sum(ev, axis=1, keepdims=True)
    idx_ref[...] = idxs

    @pl.when(pid == 0)
    def _init():
        imp_acc[...] = jnp.zeros_like(imp_acc)
        load_acc[...] = jnp.zeros_like(load_acc)

    imp_acc[...] += jnp.sum(probs, axis=0, keepdims=True)
    load_acc[...] += jnp.sum(sel, axis=0, keepdims=True)

    @pl.when(pid == grid - 1)
    def _finish():
        imp = imp_acc[...] / jnp.float32(n_tokens)
        load = load_acc[...] / jnp.float32(n_tokens * TOPK)
        loss_ref[...] = jnp.float32(e_dim) * jnp.sum(
            imp * load, axis=(0, 1), keepdims=True)


def kernel(hidden_states, gate_weight):
    n_tokens, d_model = hidden_states.shape
    e_dim = gate_weight.shape[0]
    bt = min(1024, n_tokens)
    grid = n_tokens // bt

    body = lambda *refs: _router_body(*refs, n_tokens=n_tokens, grid=grid)
    out = pl.pallas_call(
        body,
        grid=(grid,),
        in_specs=[
            pl.BlockSpec((bt, d_model), lambda i: (i, 0)),
            pl.BlockSpec((d_model, e_dim), lambda i: (0, 0)),
        ],
        out_specs=[
            pl.BlockSpec((bt, TOPK), lambda i: (i, 0)),
            pl.BlockSpec((bt, TOPK), lambda i: (i, 0)),
            pl.BlockSpec((bt, e_dim), lambda i: (i, 0)),
            pl.BlockSpec((1, 1), lambda i: (0, 0)),
        ],
        out_shape=[
            jax.ShapeDtypeStruct((n_tokens, TOPK), jnp.int32),
            jax.ShapeDtypeStruct((n_tokens, TOPK), jnp.float32),
            jax.ShapeDtypeStruct((n_tokens, e_dim), jnp.float32),
            jax.ShapeDtypeStruct((1, 1), jnp.float32),
        ],
        scratch_shapes=[
            pltpu.VMEM((1, e_dim), jnp.float32),
            pltpu.VMEM((1, e_dim), jnp.float32),
        ],
        compiler_params=pltpu.CompilerParams(
            dimension_semantics=("arbitrary",),
        ),
    )(hidden_states, gate_weight.T)
    idxs, weights, logits, loss = out
    return idxs, weights, logits, loss[0, 0]


# speedup vs baseline: 1.5091x; 1.0153x over previous
import jax
import jax.numpy as jnp
from jax.experimental import pallas as pl
from jax.experimental.pallas import tpu as pltpu

TOPK = 8


def _router_body(x1_ref, x2_ref, wt_ref, idx_ref, w_ref, logits_ref, loss_ref,
                 imp_acc, load_acc, *, n_tokens, grid):
    pid = pl.program_id(0)
    e_dim = wt_ref.shape[1]
    bt = x1_ref.shape[0]
    dh = x1_ref.shape[1]

    logits = jnp.dot(x1_ref[...], wt_ref[:dh, :],
                     preferred_element_type=jnp.float32)
    logits += jnp.dot(x2_ref[...], wt_ref[dh:, :],
                      preferred_element_type=jnp.float32)
    logits_ref[...] = logits

    m = jnp.max(logits, axis=1, keepdims=True)
    ex = jnp.exp(logits - m)
    s = jnp.sum(ex, axis=1, keepdims=True)
    probs = ex / s

    lane = jax.lax.broadcasted_iota(jnp.int32, (bt, e_dim), 1)
    work = logits
    sel = jnp.zeros((bt, e_dim), jnp.float32)
    top_v = []
    top_i = []
    for _ in range(TOPK):
        cm = jnp.max(work, axis=1, keepdims=True)
        ci = jnp.min(jnp.where(work == cm, lane, e_dim), axis=1, keepdims=True)
        chosen = lane == ci
        sel = jnp.where(chosen, 1.0, sel)
        work = jnp.where(chosen, -jnp.inf, work)
        top_v.append(cm)
        top_i.append(ci)
    vals = jnp.concatenate(top_v, axis=1)
    idxs = jnp.concatenate(top_i, axis=1)

    ev = jnp.exp(vals - m)
    w_ref[...] = ev / jnp.sum(ev, axis=1, keepdims=True)
    idx_ref[...] = idxs

    @pl.when(pid == 0)
    def _init():
        imp_acc[...] = jnp.zeros_like(imp_acc)
        load_acc[...] = jnp.zeros_like(load_acc)

    imp_acc[...] += jnp.sum(probs, axis=0, keepdims=True)
    load_acc[...] += jnp.sum(sel, axis=0, keepdims=True)

    @pl.when(pid == grid - 1)
    def _finish():
        imp = imp_acc[...] / jnp.float32(n_tokens)
        load = load_acc[...] / jnp.float32(n_tokens * TOPK)
        loss_ref[...] = jnp.float32(e_dim) * jnp.sum(
            imp * load, axis=(0, 1), keepdims=True)


def kernel(hidden_states, gate_weight):
    n_tokens, d_model = hidden_states.shape
    e_dim = gate_weight.shape[0]
    bt = min(1024, n_tokens)
    grid = n_tokens // bt

    body = lambda *refs: _router_body(*refs, n_tokens=n_tokens, grid=grid)
    out = pl.pallas_call(
        body,
        grid=(grid,),
        in_specs=[
            pl.BlockSpec((bt, d_model // 2), lambda i: (i, 0)),
            pl.BlockSpec((bt, d_model // 2), lambda i: (i, 1)),
            pl.BlockSpec((d_model, e_dim), lambda i: (0, 0)),
        ],
        out_specs=[
            pl.BlockSpec((bt, TOPK), lambda i: (i, 0)),
            pl.BlockSpec((bt, TOPK), lambda i: (i, 0)),
            pl.BlockSpec((bt, e_dim), lambda i: (i, 0)),
            pl.BlockSpec((1, 1), lambda i: (0, 0)),
        ],
        out_shape=[
            jax.ShapeDtypeStruct((n_tokens, TOPK), jnp.int32),
            jax.ShapeDtypeStruct((n_tokens, TOPK), jnp.float32),
            jax.ShapeDtypeStruct((n_tokens, e_dim), jnp.float32),
            jax.ShapeDtypeStruct((1, 1), jnp.float32),
        ],
        scratch_shapes=[
            pltpu.VMEM((1, e_dim), jnp.float32),
            pltpu.VMEM((1, e_dim), jnp.float32),
        ],
        compiler_params=pltpu.CompilerParams(
            dimension_semantics=("arbitrary",),
        ),
    )(hidden_states, hidden_states, gate_weight.T)
    idxs, weights, logits, loss = out
    return idxs, weights, logits, loss[0, 0]
